# Initial kernel scaffold; baseline (speedup 1.0000x reference)
import jax
import jax.numpy as jnp
from jax.experimental import pallas as pl


def kernel(stims, labels, n_ref_games):
    raise NotImplementedError("write your pallas kernel here")


# speedup vs baseline: 3.4253x; 3.4253x over previous
import functools

import jax
import jax.numpy as jnp
import numpy as np
from jax import lax
from jax.experimental import pallas as pl
from jax.experimental.pallas import tpu as pltpu
from jax.experimental.pallas import tpu_sc as plsc

_B, _N, _F, _R = 4096, 50, 128, 8
_JT = _B * _R
_TBC = 256
_DBC = 128
_NTB = _B // _TBC
_NDB = _B // _DBC

_TINY = np.float32(np.finfo(np.float32).tiny)


def _threefry_bits(cnt, ks0, ks1, ks2):
    rot = ((13, 15, 26, 6), (17, 29, 16, 24))
    x0 = jnp.zeros_like(cnt) + ks0
    x1 = cnt + ks1
    ks = (ks0, ks1, ks2)
    for i in range(5):
        for r in rot[i % 2]:
            x0 = x0 + x1
            x1 = (x1 << jnp.uint32(r)) | (x1 >> jnp.uint32(32 - r))
            x1 = x1 ^ x0
        x0 = x0 + ks[(i + 1) % 3]
        x1 = x1 + ks[(i + 2) % 3] + jnp.uint32(i + 1)
    return x0 ^ x1


def _gumbel(cnt, ks0, ks1, ks2):
    bits = _threefry_bits(cnt, ks0, ks1, ks2)
    fb = (bits >> jnp.uint32(9)) | jnp.uint32(0x3F800000)
    f = lax.bitcast_convert_type(fb, jnp.float32) - jnp.float32(1.0)
    u = jnp.maximum(_TINY, f * (jnp.float32(1.0) - _TINY) + _TINY)
    return -jnp.log(-jnp.log(u))


def _argmax_ax1(scores, draws, cols):
    m = jnp.max(scores, axis=1, keepdims=True)
    nidx = lax.broadcasted_iota(jnp.int32, (draws, _N, cols), 1)
    return jnp.min(jnp.where(scores == m, nidx, _N), axis=1)


def _target_body(keys_ref, zf_ref, labt_ref, out_ref):
    pid = pl.program_id(0)
    ks0, ks1 = keys_ref[0], keys_ref[1]
    ks2 = ks0 ^ ks1 ^ jnp.uint32(0x1BD11BDA)

    rr = lax.broadcasted_iota(jnp.uint32, (_R, _N, _TBC), 0)
    nn = lax.broadcasted_iota(jnp.uint32, (_R, _N, _TBC), 1)
    bl = lax.broadcasted_iota(jnp.uint32, (_R, _N, _TBC), 2)
    b = jnp.uint32(pid * _TBC) + bl
    cnt = (b * jnp.uint32(_R) + rr) * jnp.uint32(_N) + nn
    g = _gumbel(cnt, ks0, ks1, ks2)

    lg = jnp.log(labt_ref[...] + jnp.float32(1e-9))
    li = _argmax_ax1(g + lg[None, :, :], _R, _TBC)

    b2 = lax.broadcasted_iota(jnp.int32, (_R, _TBC), 1) + pid * _TBC
    out_ref[0] = b2 * _N + li + zf_ref[0]


def _distr_body(keys_ref, labt_ref, out_ref):
    pid = pl.program_id(0)
    ks0, ks1 = keys_ref[2], keys_ref[3]
    ks2 = ks0 ^ ks1 ^ jnp.uint32(0x1BD11BDA)

    sr = lax.broadcasted_iota(jnp.uint32, (2 * _R, _N, _DBC), 0)
    nn = lax.broadcasted_iota(jnp.uint32, (2 * _R, _N, _DBC), 1)
    bl = lax.broadcasted_iota(jnp.uint32, (2 * _R, _N, _DBC), 2)
    s = jnp.uint32(2) * (sr & jnp.uint32(7)) + (sr >> jnp.uint32(3))
    b = jnp.uint32(pid * _DBC) + bl
    cnt = (b * jnp.uint32(2 * _R) + s) * jnp.uint32(_N) + nn
    g = _gumbel(cnt, ks0, ks1, ks2)

    lg = jnp.log(jnp.float32(1.0) - labt_ref[...] + jnp.float32(1e-9))
    li = _argmax_ax1(g + lg[None, :, :], 2 * _R, _DBC)

    b2 = lax.broadcasted_iota(jnp.int32, (2 * _R, _DBC), 1) + pid * _DBC
    out_ref[0] = b2 * _N + li


def _sample_indices(labels_t, keys, zf, interpret=False):
    idx_t = pl.pallas_call(
        _target_body,
        grid=(_NTB,),
        in_specs=[
            pl.BlockSpec(memory_space=pltpu.SMEM),
            pl.BlockSpec(memory_space=pltpu.SMEM),
            pl.BlockSpec((_N, _TBC), lambda i: (0, i)),
        ],
        out_specs=pl.BlockSpec((1, _R, _TBC), lambda i: (i, 0, 0)),
        out_shape=jax.ShapeDtypeStruct((_NTB, _R, _TBC), jnp.int32),
        interpret=interpret,
    )(keys, zf, labels_t)
    idx_d = pl.pallas_call(
        _distr_body,
        grid=(_NDB,),
        in_specs=[
            pl.BlockSpec(memory_space=pltpu.SMEM),
            pl.BlockSpec((_N, _DBC), lambda i: (0, i)),
        ],
        out_specs=pl.BlockSpec((1, 2 * _R, _DBC), lambda i: (i, 0, 0)),
        out_shape=jax.ShapeDtypeStruct((_NDB, 2 * _R, _DBC), jnp.int32),
        interpret=interpret,
    )(keys, labels_t)
    return idx_t.reshape(-1), idx_d.reshape(-1)


_NW = 32
_GCH = 128


def _sc_gather(table, idx_t, idx_d):
    mesh = plsc.VectorSubcoreMesh(core_axis_name="c", subcore_axis_name="s")

    @functools.partial(
        pl.kernel,
        mesh=mesh,
        out_type=jax.ShapeDtypeStruct((_B, _R, 3, _F), jnp.float32),
        scratch_types=[
            pltpu.VMEM((2 * _R * _GCH,), jnp.int32),
            pltpu.VMEM((_R * _TBC,), jnp.int32),
            pltpu.VMEM((_GCH, 1, 1, _F), jnp.float32),
            pltpu.SemaphoreType.DMA,
        ],
    )
    def k(table_hbm, t_hbm, d_hbm, out_hbm, d_v, t_v, rows_v, sem):
        wid = lax.axis_index("s") * 2 + lax.axis_index("c")
        b0 = wid * _DBC
        tchunk = wid // 2
        thalf = wid - tchunk * 2
        pltpu.sync_copy(d_hbm.at[pl.ds(wid * (2 * _R) * _DBC, 2 * _R * _GCH)], d_v)
        pltpu.sync_copy(t_hbm.at[pl.ds(tchunk * _R * _TBC, _R * _TBC)], t_v)

        def put(idx_ref, off, r, slot):
            pltpu.async_copy(
                table_hbm.at[idx_ref.at[pl.ds(off, _GCH)]], rows_v, sem
            ).wait()
            pltpu.sync_copy(
                rows_v,
                out_hbm.at[pl.ds(b0, _GCH), pl.ds(r, 1), pl.ds(slot, 1)],
            )

        for r in range(_R):
            put(t_v, r * _TBC + thalf * _GCH, r, 0)
            put(d_v, r * _DBC, r, 1)
            put(d_v, (_R + r) * _DBC, r, 2)

    return k(table, idx_t, idx_d)


def kernel(stims, labels, n_ref_games):
    bsize, n_stims, n_feats = stims.shape
    zf = jnp.reshape((jnp.asarray(n_ref_games) - _R).astype(jnp.int32), (1,))
    kd = jax.random.key_data(jax.random.split(jax.random.key(42)))
    keys = kd.reshape(4).astype(jnp.uint32)
    idx_t, idx_d = _sample_indices(labels.T, keys, zf)
    table = stims.reshape(bsize * n_stims, 1, 1, n_feats)
    out = _sc_gather(table, idx_t, idx_d)
    return out.reshape(bsize * _R, 3, n_feats)


# speedup vs baseline: 3.6058x; 1.0527x over previous
import functools

import jax
import jax.numpy as jnp
import numpy as np
from jax import lax
from jax.experimental import pallas as pl
from jax.experimental.pallas import tpu as pltpu
from jax.experimental.pallas import tpu_sc as plsc

_B, _N, _F, _R = 4096, 50, 128, 8
_JT = _B * _R
_TBC = 256
_DBC = 128
_NTB = _B // _TBC
_NDB = _B // _DBC

_TINY = np.float32(np.finfo(np.float32).tiny)


def _threefry_bits(cnt, ks0, ks1, ks2):
    rot = ((13, 15, 26, 6), (17, 29, 16, 24))
    x0 = jnp.zeros_like(cnt) + ks0
    x1 = cnt + ks1
    ks = (ks0, ks1, ks2)
    for i in range(5):
        for r in rot[i % 2]:
            x0 = x0 + x1
            x1 = (x1 << jnp.uint32(r)) | (x1 >> jnp.uint32(32 - r))
            x1 = x1 ^ x0
        x0 = x0 + ks[(i + 1) % 3]
        x1 = x1 + ks[(i + 2) % 3] + jnp.uint32(i + 1)
    return x0 ^ x1


def _gumbel(cnt, ks0, ks1, ks2):
    bits = _threefry_bits(cnt, ks0, ks1, ks2)
    fb = (bits >> jnp.uint32(9)) | jnp.uint32(0x3F800000)
    f = lax.bitcast_convert_type(fb, jnp.float32) - jnp.float32(1.0)
    u = jnp.maximum(_TINY, f * (jnp.float32(1.0) - _TINY) + _TINY)
    return -jnp.log(-jnp.log(u))


def _argmax_ax1(scores, draws, cols):
    m = jnp.max(scores, axis=1, keepdims=True)
    nidx = lax.broadcasted_iota(jnp.int32, (draws, _N, cols), 1)
    return jnp.min(jnp.where(scores == m, nidx, _N), axis=1)


def _target_body(keys_ref, zf_ref, lab_ref, out_ref):
    pid = pl.program_id(0)
    ks0, ks1 = keys_ref[0], keys_ref[1]
    ks2 = ks0 ^ ks1 ^ jnp.uint32(0x1BD11BDA)

    rr = lax.broadcasted_iota(jnp.uint32, (_R, _N, _TBC), 0)
    nn = lax.broadcasted_iota(jnp.uint32, (_R, _N, _TBC), 1)
    bl = lax.broadcasted_iota(jnp.uint32, (_R, _N, _TBC), 2)
    b = jnp.uint32(pid * _TBC) + bl
    cnt = (b * jnp.uint32(_R) + rr) * jnp.uint32(_N) + nn
    g = _gumbel(cnt, ks0, ks1, ks2)

    labt = lax.transpose(lab_ref[...], (1, 0))
    lg = jnp.log(labt + jnp.float32(1e-9))
    li = _argmax_ax1(g + lg[None, :, :], _R, _TBC)

    b2 = lax.broadcasted_iota(jnp.int32, (_R, _TBC), 1) + pid * _TBC
    out_ref[0] = b2 * _N + li + zf_ref[0]


def _distr_body(keys_ref, lab_ref, out_ref):
    pid = pl.program_id(0)
    ks0, ks1 = keys_ref[2], keys_ref[3]
    ks2 = ks0 ^ ks1 ^ jnp.uint32(0x1BD11BDA)

    sr = lax.broadcasted_iota(jnp.uint32, (2 * _R, _N, _DBC), 0)
    nn = lax.broadcasted_iota(jnp.uint32, (2 * _R, _N, _DBC), 1)
    bl = lax.broadcasted_iota(jnp.uint32, (2 * _R, _N, _DBC), 2)
    s = jnp.uint32(2) * (sr & jnp.uint32(7)) + (sr >> jnp.uint32(3))
    b = jnp.uint32(pid * _DBC) + bl
    cnt = (b * jnp.uint32(2 * _R) + s) * jnp.uint32(_N) + nn
    g = _gumbel(cnt, ks0, ks1, ks2)

    labt = lax.transpose(lab_ref[...], (1, 0))
    lg = jnp.log(jnp.float32(1.0) - labt + jnp.float32(1e-9))
    li = _argmax_ax1(g + lg[None, :, :], 2 * _R, _DBC)

    b2 = lax.broadcasted_iota(jnp.int32, (2 * _R, _DBC), 1) + pid * _DBC
    out_ref[0] = b2 * _N + li


def _sample_indices(labels, keys, zf, interpret=False):
    idx_t = pl.pallas_call(
        _target_body,
        grid=(_NTB,),
        in_specs=[
            pl.BlockSpec(memory_space=pltpu.SMEM),
            pl.BlockSpec(memory_space=pltpu.SMEM),
            pl.BlockSpec((_TBC, _N), lambda i: (i, 0)),
        ],
        out_specs=pl.BlockSpec((1, _R, _TBC), lambda i: (i, 0, 0)),
        out_shape=jax.ShapeDtypeStruct((_NTB, _R, _TBC), jnp.int32),
        interpret=interpret,
    )(keys, zf, labels)
    idx_d = pl.pallas_call(
        _distr_body,
        grid=(_NDB,),
        in_specs=[
            pl.BlockSpec(memory_space=pltpu.SMEM),
            pl.BlockSpec((_DBC, _N), lambda i: (i, 0)),
        ],
        out_specs=pl.BlockSpec((1, 2 * _R, _DBC), lambda i: (i, 0, 0)),
        out_shape=jax.ShapeDtypeStruct((_NDB, 2 * _R, _DBC), jnp.int32),
        interpret=interpret,
    )(keys, labels)
    return idx_t.reshape(-1), idx_d.reshape(-1)


_NW = 32
_GCH = 128


def _sc_gather(table, idx_t, idx_d):
    mesh = plsc.VectorSubcoreMesh(core_axis_name="c", subcore_axis_name="s")

    @functools.partial(
        pl.kernel,
        mesh=mesh,
        out_type=jax.ShapeDtypeStruct((_B, _R, 3, _F), jnp.float32),
        scratch_types=[
            pltpu.VMEM((2 * _R * _GCH,), jnp.int32),
            pltpu.VMEM((_R * _TBC,), jnp.int32),
            pltpu.VMEM((_GCH, 1, 1, _F), jnp.float32),
            pltpu.VMEM((_GCH, 1, 1, _F), jnp.float32),
            pltpu.SemaphoreType.DMA,
            pltpu.SemaphoreType.DMA,
            pltpu.SemaphoreType.DMA,
            pltpu.SemaphoreType.DMA,
        ],
    )
    def k(table_hbm, t_hbm, d_hbm, out_hbm, d_v, t_v, rows_a, rows_b,
          gs_a, gs_b, ws_a, ws_b):
        wid = lax.axis_index("s") * 2 + lax.axis_index("c")
        b0 = wid * _DBC
        tchunk = wid // 2
        thalf = wid - tchunk * 2
        pltpu.sync_copy(d_hbm.at[pl.ds(wid * (2 * _R) * _DBC, 2 * _R * _GCH)], d_v)
        pltpu.sync_copy(t_hbm.at[pl.ds(tchunk * _R * _TBC, _R * _TBC)], t_v)

        chunks = []
        for r in range(_R):
            chunks.append((t_v, r * _TBC + thalf * _GCH, r, 0))
            chunks.append((d_v, r * _DBC, r, 1))
            chunks.append((d_v, (_R + r) * _DBC, r, 2))

        bufs = (rows_a, rows_b)
        gsem = (gs_a, gs_b)
        wsem = (ws_a, ws_b)

        def start_gather(i):
            idx_ref, off, _, _ = chunks[i]
            return pltpu.async_copy(
                table_hbm.at[idx_ref.at[pl.ds(off, _GCH)]],
                bufs[i % 2], gsem[i % 2],
            )

        def start_write(i):
            _, _, r, slot = chunks[i]
            return pltpu.async_copy(
                bufs[i % 2],
                out_hbm.at[pl.ds(b0, _GCH), pl.ds(r, 1), pl.ds(slot, 1)],
                wsem[i % 2],
            )

        n = len(chunks)
        g = [None] * n
        w = [None] * n
        g[0] = start_gather(0)
        for i in range(n):
            if i >= 1:
                w[i - 1].wait()
            if i < n - 1:
                g[i + 1] = start_gather(i + 1)
            g[i].wait()
            w[i] = start_write(i)
        w[n - 1].wait()

    return k(table, idx_t, idx_d)


def kernel(stims, labels, n_ref_games):
    bsize, n_stims, n_feats = stims.shape
    zf = jnp.reshape((jnp.asarray(n_ref_games) - _R).astype(jnp.int32), (1,))
    kd = jax.random.key_data(jax.random.split(jax.random.key(42)))
    keys = kd.reshape(4).astype(jnp.uint32)
    idx_t, idx_d = _sample_indices(labels, keys, zf)
    table = stims.reshape(bsize * n_stims, 1, 1, n_feats)
    out = _sc_gather(table, idx_t, idx_d)
    return out.reshape(bsize * _R, 3, n_feats)


# speedup vs baseline: 3.8964x; 1.0806x over previous
import functools

import jax
import jax.numpy as jnp
import numpy as np
from jax import lax
from jax.experimental import pallas as pl
from jax.experimental.pallas import tpu as pltpu
from jax.experimental.pallas import tpu_sc as plsc

_B, _N, _F, _R = 4096, 50, 128, 8
_NP = 56
_JT = _B * _R
_TBC = 256
_DBC = 128
_NTB = _B // _TBC
_NDB = _B // _DBC

_TINY = np.float32(np.finfo(np.float32).tiny)


def _threefry_bits(cnt, ks0, ks1, ks2):
    rot = ((13, 15, 26, 6), (17, 29, 16, 24))
    x0 = jnp.zeros_like(cnt) + ks0
    x1 = cnt + ks1
    ks = (ks0, ks1, ks2)
    for i in range(5):
        for r in rot[i % 2]:
            x0 = x0 + x1
            x1 = (x1 << jnp.uint32(r)) | (x1 >> jnp.uint32(32 - r))
            x1 = x1 ^ x0
        x0 = x0 + ks[(i + 1) % 3]
        x1 = x1 + ks[(i + 2) % 3] + jnp.uint32(i + 1)
    return x0 ^ x1


def _gumbel(cnt, ks0, ks1, ks2):
    bits = _threefry_bits(cnt, ks0, ks1, ks2)
    fb = (bits >> jnp.uint32(9)) | jnp.uint32(0x3F800000)
    f = lax.bitcast_convert_type(fb, jnp.float32) - jnp.float32(1.0)
    u = jnp.maximum(_TINY, f * (jnp.float32(1.0) - _TINY) + _TINY)
    return -jnp.log(-jnp.log(u))


def _argmax_ax1(scores, draws, cols):
    m = jnp.max(scores, axis=1, keepdims=True)
    nidx = lax.broadcasted_iota(jnp.int32, (draws, _N, cols), 1)
    return jnp.min(jnp.where(scores == m, nidx, _N), axis=1)


def _target_body(keys_ref, zf_ref, lab_ref, out_ref):
    pid = pl.program_id(0)
    ks0, ks1 = keys_ref[0], keys_ref[1]
    ks2 = ks0 ^ ks1 ^ jnp.uint32(0x1BD11BDA)

    rr = lax.broadcasted_iota(jnp.uint32, (_R, _N, _TBC), 0)
    nn = lax.broadcasted_iota(jnp.uint32, (_R, _N, _TBC), 1)
    bl = lax.broadcasted_iota(jnp.uint32, (_R, _N, _TBC), 2)
    b = jnp.uint32(pid * _TBC) + bl
    cnt = (b * jnp.uint32(_R) + rr) * jnp.uint32(_N) + nn
    g = _gumbel(cnt, ks0, ks1, ks2)

    labt = lax.transpose(lab_ref[...], (1, 0))
    lg = jnp.log(labt + jnp.float32(1e-9))
    li = _argmax_ax1(g + lg[None, :, :], _R, _TBC)

    b2 = lax.broadcasted_iota(jnp.int32, (_R, _TBC), 1) + pid * _TBC
    out_ref[0] = b2 * _NP + li + zf_ref[0]


def _distr_body(keys_ref, lab_ref, stims_ref, out_ref, table_ref):
    pid = pl.program_id(0)
    ks0, ks1 = keys_ref[2], keys_ref[3]
    ks2 = ks0 ^ ks1 ^ jnp.uint32(0x1BD11BDA)

    sr = lax.broadcasted_iota(jnp.uint32, (2 * _R, _N, _DBC), 0)
    nn = lax.broadcasted_iota(jnp.uint32, (2 * _R, _N, _DBC), 1)
    bl = lax.broadcasted_iota(jnp.uint32, (2 * _R, _N, _DBC), 2)
    s = jnp.uint32(2) * (sr & jnp.uint32(7)) + (sr >> jnp.uint32(3))
    b = jnp.uint32(pid * _DBC) + bl
    cnt = (b * jnp.uint32(2 * _R) + s) * jnp.uint32(_N) + nn
    g = _gumbel(cnt, ks0, ks1, ks2)

    labt = lax.transpose(lab_ref[...], (1, 0))
    lg = jnp.log(jnp.float32(1.0) - labt + jnp.float32(1e-9))
    li = _argmax_ax1(g + lg[None, :, :], 2 * _R, _DBC)

    b2 = lax.broadcasted_iota(jnp.int32, (2 * _R, _DBC), 1) + pid * _DBC
    out_ref[0] = b2 * _NP + li
    table_ref[:, : _N, :] = stims_ref[...]


def _sample_indices(labels, stims, keys, zf, interpret=False):
    idx_t = pl.pallas_call(
        _target_body,
        grid=(_NTB,),
        in_specs=[
            pl.BlockSpec(memory_space=pltpu.SMEM),
            pl.BlockSpec(memory_space=pltpu.SMEM),
            pl.BlockSpec((_TBC, _N), lambda i: (i, 0)),
        ],
        out_specs=pl.BlockSpec((1, _R, _TBC), lambda i: (i, 0, 0)),
        out_shape=jax.ShapeDtypeStruct((_NTB, _R, _TBC), jnp.int32),
        interpret=interpret,
    )(keys, zf, labels)
    idx_d, table = pl.pallas_call(
        _distr_body,
        grid=(_NDB,),
        in_specs=[
            pl.BlockSpec(memory_space=pltpu.SMEM),
            pl.BlockSpec((_DBC, _N), lambda i: (i, 0)),
            pl.BlockSpec((_DBC, _N, _F), lambda i: (i, 0, 0)),
        ],
        out_specs=[
            pl.BlockSpec((1, 2 * _R, _DBC), lambda i: (i, 0, 0)),
            pl.BlockSpec((_DBC, _NP, _F), lambda i: (i, 0, 0)),
        ],
        out_shape=[
            jax.ShapeDtypeStruct((_NDB, 2 * _R, _DBC), jnp.int32),
            jax.ShapeDtypeStruct((_B, _NP, _F), jnp.float32),
        ],
        interpret=interpret,
    )(keys, labels, stims)
    return idx_t.reshape(-1), idx_d.reshape(-1), table


_NW = 32
_GCH = 128


def _sc_gather(table, idx_t, idx_d):
    mesh = plsc.VectorSubcoreMesh(core_axis_name="c", subcore_axis_name="s")

    @functools.partial(
        pl.kernel,
        mesh=mesh,
        out_type=jax.ShapeDtypeStruct((_B, _R, 3, _F), jnp.float32),
        scratch_types=[
            pltpu.VMEM((2 * _R * _GCH,), jnp.int32),
            pltpu.VMEM((_R * _TBC,), jnp.int32),
            pltpu.VMEM((_GCH, _F), jnp.float32),
            pltpu.VMEM((_GCH, _F), jnp.float32),
            pltpu.SemaphoreType.DMA,
            pltpu.SemaphoreType.DMA,
            pltpu.SemaphoreType.DMA,
            pltpu.SemaphoreType.DMA,
        ],
    )
    def k(table_hbm, t_hbm, d_hbm, out_hbm, d_v, t_v, rows_a, rows_b,
          gs_a, gs_b, ws_a, ws_b):
        wid = lax.axis_index("s") * 2 + lax.axis_index("c")
        b0 = wid * _DBC
        tchunk = wid // 2
        thalf = wid - tchunk * 2
        pltpu.sync_copy(d_hbm.at[pl.ds(wid * (2 * _R) * _DBC, 2 * _R * _GCH)], d_v)
        pltpu.sync_copy(t_hbm.at[pl.ds(tchunk * _R * _TBC, _R * _TBC)], t_v)

        chunks = []
        for r in range(_R):
            chunks.append((t_v, r * _TBC + thalf * _GCH, r, 0))
            chunks.append((d_v, r * _DBC, r, 1))
            chunks.append((d_v, (_R + r) * _DBC, r, 2))

        bufs = (rows_a, rows_b)
        gsem = (gs_a, gs_b)
        wsem = (ws_a, ws_b)

        def start_gather(i):
            idx_ref, off, _, _ = chunks[i]
            return pltpu.async_copy(
                table_hbm.at[idx_ref.at[pl.ds(off, _GCH)]],
                bufs[i % 2], gsem[i % 2],
            )

        def start_write(i):
            _, _, r, slot = chunks[i]
            return pltpu.async_copy(
                bufs[i % 2].reshape(_GCH, 1, 1, _F),
                out_hbm.at[pl.ds(b0, _GCH), pl.ds(r, 1), pl.ds(slot, 1)],
                wsem[i % 2],
            )

        n = len(chunks)
        g = [None] * n
        w = [None] * n
        g[0] = start_gather(0)
        for i in range(n):
            if i >= 1:
                w[i - 1].wait()
            if i < n - 1:
                g[i + 1] = start_gather(i + 1)
            g[i].wait()
            w[i] = start_write(i)
        w[n - 1].wait()

    return k(table, idx_t, idx_d)


def kernel(stims, labels, n_ref_games):
    bsize, n_stims, n_feats = stims.shape
    zf = jnp.reshape((jnp.asarray(n_ref_games) - _R).astype(jnp.int32), (1,))
    kd = jax.random.key_data(jax.random.split(jax.random.key(42)))
    keys = kd.reshape(4).astype(jnp.uint32)
    idx_t, idx_d, table3 = _sample_indices(labels, stims, keys, zf)
    out = _sc_gather(table3.reshape(bsize * _NP, n_feats), idx_t, idx_d)
    return out.reshape(bsize * _R, 3, n_feats)
